# baseline (speedup 1.0000x reference)
import functools

import jax
import jax.numpy as jnp
from jax import lax
from jax.experimental import pallas as pl
from jax.experimental.pallas import tpu as pltpu
from jax.experimental.pallas import tpu_sc as plsc

N = 100000
F = 16
NS, CB = 16, 512
NP = 100224
RPT = NP // NS
TRASH = N
EP = 3276800
STEPS = (EP // CB) // NS
BN = 2000


def _make_segsum():
    mesh = plsc.VectorSubcoreMesh(
        core_axis_name="c", subcore_axis_name="s", num_cores=1, num_subcores=NS
    )

    @functools.partial(
        pl.kernel,
        out_type=jax.ShapeDtypeStruct((NP, F), jnp.float32),
        mesh=mesh,
        scratch_types=[
            pltpu.VMEM((3, CB), jnp.int32),
            pltpu.VMEM((3, CB), jnp.int32),
            pltpu.VMEM((2, CB, F), jnp.float32),
            pltpu.VMEM_SHARED((NP, F), jnp.float32),
            pltpu.SemaphoreType.DMA,
            pltpu.SemaphoreType.DMA,
            pltpu.SemaphoreType.DMA,
        ],
        compiler_params=pltpu.CompilerParams(use_tc_tiling_on_sc=False),
    )
    def segsum(h_hbm, src_hbm, dst_hbm, zero_hbm, out_hbm, sidx, didx, rows,
               agg, gsem, ssem, isem):
        s = lax.axis_index("s")

        pltpu.sync_copy(zero_hbm.at[pl.ds(s * RPT, RPT)],
                        agg.at[pl.ds(s * RPT, RPT)])
        plsc.subcore_barrier()

        steps = STEPS
        base = s * STEPS

        pltpu.async_copy(src_hbm.at[base], sidx.at[0], isem)
        pltpu.async_copy(dst_hbm.at[base], didx.at[0], isem)

        def _iter(g, carry):
            i0 = g % 3
            i1 = (g + 1) % 3
            im = (g - 1) % 3
            p = g % 2
            q = (g + 1) % 2

            @pl.when(g >= 2)
            def _():
                pltpu.make_async_copy(
                    rows.at[p], agg.at[didx.at[i1]], ssem
                ).wait()

            @pl.when(g < steps)
            def _():
                pltpu.make_async_copy(src_hbm.at[base + g], sidx.at[i0], isem).wait()
                pltpu.make_async_copy(dst_hbm.at[base + g], didx.at[i0], isem).wait()
                pltpu.async_copy(h_hbm.at[sidx.at[i0]], rows.at[p], gsem)

            @pl.when(g + 1 < steps)
            def _():
                pltpu.async_copy(src_hbm.at[base + g + 1], sidx.at[i1], isem)
                pltpu.async_copy(dst_hbm.at[base + g + 1], didx.at[i1], isem)

            @pl.when(jnp.logical_and(g >= 1, g - 1 < steps))
            def _():
                pltpu.make_async_copy(
                    h_hbm.at[sidx.at[im]], rows.at[q], gsem
                ).wait()
                pltpu.async_copy(rows.at[q], agg.at[didx.at[im]], ssem, add=True)

            return carry

        lax.fori_loop(0, steps + 2, _iter, 0)
        plsc.subcore_barrier()
        pltpu.sync_copy(
            agg.at[pl.ds(s * RPT, RPT)], out_hbm.at[pl.ds(s * RPT, RPT)]
        )

    return segsum


_SEGSUM = _make_segsum()

_ROW = lambda i: (i, 0)
_FIX = lambda i: (0, 0)


def _tc1_body(x_ref, w_ref, b_ref, h_ref):
    h_ref[...] = jax.nn.relu(
        jnp.dot(x_ref[...], w_ref[...], preferred_element_type=jnp.float32)
        + b_ref[...]
    )


def _tc1(xp, w, b):
    return pl.pallas_call(
        _tc1_body,
        grid=(N // BN,),
        in_specs=[
            pl.BlockSpec((BN, F), _ROW),
            pl.BlockSpec((F, F), _FIX),
            pl.BlockSpec((1, F), _FIX),
        ],
        out_specs=pl.BlockSpec((BN, F), _ROW),
        out_shape=jax.ShapeDtypeStruct((N, F), jnp.float32),
    )(xp, w, b)


def _tc2_body(a0, xr, wl, bl, wr, wp, bp, x2_ref, h2_ref):
    agg = a0[...]
    x2 = jax.nn.relu(
        jnp.dot(agg, wl[...], preferred_element_type=jnp.float32)
        + bl[...]
        + jnp.dot(xr[...], wr[...], preferred_element_type=jnp.float32)
    )
    x2_ref[...] = x2
    h2_ref[...] = jax.nn.relu(
        jnp.dot(x2, wp[...], preferred_element_type=jnp.float32) + bp[...]
    )


def _tc2(a0, xr, wl, bl, wr, wp, bp):
    return pl.pallas_call(
        _tc2_body,
        grid=(N // BN,),
        in_specs=[
            pl.BlockSpec((BN, F), _ROW),
            pl.BlockSpec((BN, F), _ROW),
            pl.BlockSpec((F, F), _FIX),
            pl.BlockSpec((1, F), _FIX),
            pl.BlockSpec((F, F), _FIX),
            pl.BlockSpec((F, F), _FIX),
            pl.BlockSpec((1, F), _FIX),
        ],
        out_specs=[pl.BlockSpec((BN, F), _ROW), pl.BlockSpec((BN, F), _ROW)],
        out_shape=[
            jax.ShapeDtypeStruct((N, F), jnp.float32),
            jax.ShapeDtypeStruct((N, F), jnp.float32),
        ],
    )(a0, xr, wl, bl, wr, wp, bp)


def _tc4_body(a0, xr, wl, bl, wr, out_ref):
    agg = a0[...]
    out_ref[...] = jax.nn.sigmoid(
        jnp.dot(agg, wl[...], preferred_element_type=jnp.float32)
        + bl[...]
        + jnp.dot(xr[...], wr[...], preferred_element_type=jnp.float32)
    )


def _tc4(a0, xr, wl, bl, wr):
    return pl.pallas_call(
        _tc4_body,
        grid=(N // BN,),
        in_specs=[
            pl.BlockSpec((BN, F), _ROW),
            pl.BlockSpec((BN, F), _ROW),
            pl.BlockSpec((F, 1), _FIX),
            pl.BlockSpec((1, 1), _FIX),
            pl.BlockSpec((F, 1), _FIX),
        ],
        out_specs=pl.BlockSpec((BN, 1), _ROW),
        out_shape=jax.ShapeDtypeStruct((N, 1), jnp.float32),
    )(a0, xr, wl, bl, wr)


def kernel(x, edge_index, Wp1, bp1, Wl1, bl1, Wr1, Wp2, bp2, Wl2, bl2, Wr2,
           Wp3, bp3, Wl3, bl3, Wr3):
    f32 = jnp.float32
    xp = jnp.zeros((N, F), f32).at[:, :3].set(x)
    wp1 = jnp.zeros((F, F), f32).at[:3, :3].set(Wp1.T)
    bp1p = jnp.zeros((1, F), f32).at[0, :3].set(bp1)
    wl1 = jnp.zeros((F, F), f32).at[:3, :].set(Wl1.T)
    bl1p = bl1.reshape(1, F)
    wr1 = jnp.zeros((F, F), f32).at[:3, :].set(Wr1.T)
    wp2, bp2p, wl2, bl2p, wr2 = Wp2.T, bp2.reshape(1, F), Wl2.T, bl2.reshape(1, F), Wr2.T
    wp3, bp3p = Wp3.T, bp3.reshape(1, F)
    wl3, bl3p, wr3 = Wl3.T, bl3.reshape(1, 1), Wr3.T

    src = edge_index[0]
    dst = edge_index[1]
    padn = EP - src.shape[0]
    src2 = jnp.concatenate([src, jnp.zeros((padn,), jnp.int32)]).reshape(EP // CB, CB)
    dst2 = jnp.concatenate([dst, jnp.full((padn,), TRASH, jnp.int32)]).reshape(EP // CB, CB)
    zf = jnp.zeros((NP, F), f32)

    h1 = _tc1(xp, wp1, bp1p)
    agg1 = _SEGSUM(h1, src2, dst2, zf)
    x2, h2 = _tc2(agg1[:N], xp, wl1, bl1p, wr1, wp2, bp2p)
    agg2 = _SEGSUM(h2, src2, dst2, zf)
    x3, h3 = _tc2(agg2[:N], x2, wl2, bl2p, wr2, wp3, bp3p)
    agg3 = _SEGSUM(h3, src2, dst2, zf)
    out = _tc4(agg3[:N], x3, wl3, bl3p, wr3)
    return out


# speedup vs baseline: 1.2467x; 1.2467x over previous
import functools

import jax
import jax.numpy as jnp
from jax import lax
from jax.experimental import pallas as pl
from jax.experimental.pallas import tpu as pltpu
from jax.experimental.pallas import tpu_sc as plsc

N = 100000
F = 16
NC, NS, CB = 2, 16, 512
NW = NC * NS
NP = 100224
RPT = NP // NS
TRASH = N
EP = 3276800
SPF = 294
SPS = 106
BN = 2000


def _make_segsum():
    mesh = plsc.VectorSubcoreMesh(
        core_axis_name="c", subcore_axis_name="s", num_cores=NC, num_subcores=NS
    )

    @functools.partial(
        pl.kernel,
        out_type=jax.ShapeDtypeStruct((NC, NP, F), jnp.float32),
        mesh=mesh,
        scratch_types=[
            pltpu.VMEM((3, CB), jnp.int32),
            pltpu.VMEM((3, CB), jnp.int32),
            pltpu.VMEM((2, CB, F), jnp.float32),
            pltpu.VMEM_SHARED((NP, F), jnp.float32),
            pltpu.SemaphoreType.DMA,
            pltpu.SemaphoreType.DMA,
            pltpu.SemaphoreType.DMA,
        ],
        compiler_params=pltpu.CompilerParams(use_tc_tiling_on_sc=False),
    )
    def segsum(h_hbm, src_hbm, dst_hbm, zero_hbm, out_hbm, sidx, didx, rows,
               agg, gsem, ssem, isem):
        c = lax.axis_index("c")
        s = lax.axis_index("s")

        pltpu.sync_copy(zero_hbm.at[pl.ds(s * RPT, RPT)],
                        agg.at[pl.ds(s * RPT, RPT)])
        plsc.subcore_barrier()

        steps = jnp.where(c == 0, SPF, SPS)
        base = jnp.where(c == 0, s * SPF, NS * SPF + s * SPS)

        pltpu.async_copy(src_hbm.at[base], sidx.at[0], isem)
        pltpu.async_copy(dst_hbm.at[base], didx.at[0], isem)

        def _iter(g, carry):
            i0 = g % 3
            i1 = (g + 1) % 3
            im = (g - 1) % 3
            p = g % 2
            q = (g + 1) % 2

            @pl.when(g >= 2)
            def _():
                pltpu.make_async_copy(
                    rows.at[p], agg.at[didx.at[i1]], ssem
                ).wait()

            @pl.when(g < steps)
            def _():
                pltpu.make_async_copy(src_hbm.at[base + g], sidx.at[i0], isem).wait()
                pltpu.make_async_copy(dst_hbm.at[base + g], didx.at[i0], isem).wait()
                pltpu.async_copy(h_hbm.at[sidx.at[i0]], rows.at[p], gsem)

            @pl.when(g + 1 < steps)
            def _():
                pltpu.async_copy(src_hbm.at[base + g + 1], sidx.at[i1], isem)
                pltpu.async_copy(dst_hbm.at[base + g + 1], didx.at[i1], isem)

            @pl.when(jnp.logical_and(g >= 1, g - 1 < steps))
            def _():
                pltpu.make_async_copy(
                    h_hbm.at[sidx.at[im]], rows.at[q], gsem
                ).wait()
                pltpu.async_copy(rows.at[q], agg.at[didx.at[im]], ssem, add=True)

            return carry

        lax.fori_loop(0, steps + 2, _iter, 0)
        plsc.subcore_barrier()
        pltpu.sync_copy(
            agg.at[pl.ds(s * RPT, RPT)], out_hbm.at[c, pl.ds(s * RPT, RPT)]
        )

    return segsum


_SEGSUM = _make_segsum()

_ROW = lambda i: (i, 0)
_FIX = lambda i: (0, 0)

NR = NP // 8
BNP = NR // 6


def _tc1_body(x_ref, w_ref, b_ref, h_ref):
    h_ref[...] = jax.nn.relu(
        jnp.dot(x_ref[...], w_ref[...], preferred_element_type=jnp.float32)
        + b_ref[...]
    )


def _tc1(xp, w, b):
    return pl.pallas_call(
        _tc1_body,
        grid=(NR // BNP,),
        in_specs=[
            pl.BlockSpec((BNP, 128), _ROW),
            pl.BlockSpec((128, 128), _FIX),
            pl.BlockSpec((1, 128), _FIX),
        ],
        out_specs=pl.BlockSpec((BNP, 128), _ROW),
        out_shape=jax.ShapeDtypeStruct((NR, 128), jnp.float32),
    )(xp, w, b)


def _tc2_body(a0, a1, xr, wl, bl, wr, wp, bp, x2_ref, h2_ref):
    agg = a0[...] + a1[...]
    x2 = jax.nn.relu(
        jnp.dot(agg, wl[...], preferred_element_type=jnp.float32)
        + bl[...]
        + jnp.dot(xr[...], wr[...], preferred_element_type=jnp.float32)
    )
    x2_ref[...] = x2
    h2_ref[...] = jax.nn.relu(
        jnp.dot(x2, wp[...], preferred_element_type=jnp.float32) + bp[...]
    )


def _tc2(a0, a1, xr, wl, bl, wr, wp, bp):
    return pl.pallas_call(
        _tc2_body,
        grid=(NR // BNP,),
        in_specs=[
            pl.BlockSpec((BNP, 128), _ROW),
            pl.BlockSpec((BNP, 128), _ROW),
            pl.BlockSpec((BNP, 128), _ROW),
            pl.BlockSpec((128, 128), _FIX),
            pl.BlockSpec((1, 128), _FIX),
            pl.BlockSpec((128, 128), _FIX),
            pl.BlockSpec((128, 128), _FIX),
            pl.BlockSpec((1, 128), _FIX),
        ],
        out_specs=[pl.BlockSpec((BNP, 128), _ROW), pl.BlockSpec((BNP, 128), _ROW)],
        out_shape=[
            jax.ShapeDtypeStruct((NR, 128), jnp.float32),
            jax.ShapeDtypeStruct((NR, 128), jnp.float32),
        ],
    )(a0, a1, xr, wl, bl, wr, wp, bp)


def _tc4_body(a0, a1, xr, wl, bl, wr, out_ref):
    agg = a0[...] + a1[...]
    out_ref[...] = jax.nn.sigmoid(
        jnp.dot(agg, wl[...], preferred_element_type=jnp.float32)
        + bl[...]
        + jnp.dot(xr[...], wr[...], preferred_element_type=jnp.float32)
    )


def _tc4(a0, a1, xr, wl, bl, wr):
    return pl.pallas_call(
        _tc4_body,
        grid=(NR // BNP,),
        in_specs=[
            pl.BlockSpec((BNP, 128), _ROW),
            pl.BlockSpec((BNP, 128), _ROW),
            pl.BlockSpec((BNP, 128), _ROW),
            pl.BlockSpec((128, 8), _FIX),
            pl.BlockSpec((1, 8), _FIX),
            pl.BlockSpec((128, 8), _FIX),
        ],
        out_specs=pl.BlockSpec((BNP, 8), _ROW),
        out_shape=jax.ShapeDtypeStruct((NR, 8), jnp.float32),
    )(a0, a1, xr, wl, bl, wr)


def kernel(x, edge_index, Wp1, bp1, Wl1, bl1, Wr1, Wp2, bp2, Wl2, bl2, Wr2,
           Wp3, bp3, Wl3, bl3, Wr3):
    f32 = jnp.float32
    eye8 = jnp.eye(8, dtype=f32)

    def kr(w):
        return jnp.kron(eye8, w)

    def b8(b):
        return jnp.tile(b.reshape(1, -1), (1, 8))

    xp = jnp.zeros((NP, F), f32).at[:N, :3].set(x).reshape(NR, 128)
    wp1 = kr(jnp.zeros((F, F), f32).at[:3, :3].set(Wp1.T))
    bp1p = b8(jnp.zeros((F,), f32).at[:3].set(bp1))
    wl1 = kr(jnp.zeros((F, F), f32).at[:3, :].set(Wl1.T))
    bl1p = b8(bl1)
    wr1 = kr(jnp.zeros((F, F), f32).at[:3, :].set(Wr1.T))
    wp2, bp2p, wl2, bl2p, wr2 = kr(Wp2.T), b8(bp2), kr(Wl2.T), b8(bl2), kr(Wr2.T)
    wp3, bp3p = kr(Wp3.T), b8(bp3)
    wl3, bl3p, wr3 = kr(Wl3.T), b8(bl3), kr(Wr3.T)

    src_ = edge_index[0]
    dst = edge_index[1]
    padn = EP - src_.shape[0]
    src2 = jnp.concatenate([src_, jnp.zeros((padn,), jnp.int32)]).reshape(EP // CB, CB)
    dst2 = jnp.concatenate([dst, jnp.full((padn,), TRASH, jnp.int32)]).reshape(EP // CB, CB)
    zf = jnp.zeros((NP, F), f32)

    def seg(h):
        a = _SEGSUM(h.reshape(NP, F), src2, dst2, zf)
        return a[0].reshape(NR, 128), a[1].reshape(NR, 128)

    h1 = _tc1(xp, wp1, bp1p)
    a0, a1 = seg(h1)
    x2, h2 = _tc2(a0, a1, xp, wl1, bl1p, wr1, wp2, bp2p)
    a0, a1 = seg(h2)
    x3, h3 = _tc2(a0, a1, x2, wl2, bl2p, wr2, wp3, bp3p)
    a0, a1 = seg(h3)
    out = _tc4(a0, a1, x3, wl3, bl3p, wr3)
    return out.reshape(NP, 1)[:N]
